# baseline (speedup 1.0000x reference)
import functools

import jax
import jax.numpy as jnp
from jax import lax
from jax.experimental import pallas as pl
from jax.experimental.pallas import tpu as pltpu
from jax.experimental.pallas import tpu_sc as plsc

_N_TOK = 16384
_D = 2048
_NE = 16
_TOK_BLK = 2048

_NW = 32
_TPW = _N_TOK // _NW
_GRP = _TPW // 16


def _logits_body(x_ref, w_ref, b_ref, o_ref):
    o_ref[...] = lax.dot_general(
        w_ref[...], x_ref[...], (((1,), (1,)), ((), ())),
        preferred_element_type=jnp.float32,
    ) + b_ref[...]


def _compute_logits_t(x, gate_w, gate_b):
    nb = _N_TOK // _TOK_BLK
    return pl.pallas_call(
        _logits_body,
        grid=(nb,),
        in_specs=[
            pl.BlockSpec((_TOK_BLK, _D), lambda i: (i, 0)),
            pl.BlockSpec((_NE, _D), lambda i: (0, 0)),
            pl.BlockSpec((_NE, 1), lambda i: (0, 0)),
        ],
        out_specs=pl.BlockSpec((_NE, _TOK_BLK), lambda i: (0, i)),
        out_shape=jax.ShapeDtypeStruct((_NE, _N_TOK), jnp.float32),
    )(x, gate_w, gate_b.reshape(_NE, 1))


def _routing_body(lt_hbm, probs_t_hbm, idx_t_hbm, lt_v, pt_v, it_v):
    c = lax.axis_index("c")
    s = lax.axis_index("s")
    wid = s * 2 + c
    base = wid * _TPW
    pltpu.sync_copy(lt_hbm.at[:, pl.ds(base, _TPW)], lt_v)

    neg_inf = jnp.full((16,), -jnp.inf, jnp.float32)

    def grp(g, carry):
        off = g * 16
        rows = [lt_v[e, pl.ds(off, 16)] for e in range(_NE)]
        m1 = rows[0]
        a1 = jnp.zeros((16,), jnp.int32)
        for e in range(1, _NE):
            upd = rows[e] > m1
            m1 = jnp.where(upd, rows[e], m1)
            a1 = jnp.where(upd, e, a1)
        m2 = neg_inf
        a2 = jnp.zeros((16,), jnp.int32)
        for e in range(_NE):
            v = jnp.where(a1 == e, neg_inf, rows[e])
            upd = v > m2
            m2 = jnp.where(upd, v, m2)
            a2 = jnp.where(upd, e, a2)
        t = jnp.exp(m2 - m1)
        denom = 1.0 + t
        p1 = 1.0 / denom
        p2 = t / denom
        off_slice = pl.ds(off, 16)
        zeros_f = jnp.zeros((16,), jnp.float32)
        for e in range(_NE):
            pt_v[e, off_slice] = (jnp.where(a1 == e, p1, zeros_f)
                                  + jnp.where(a2 == e, p2, zeros_f))
        it_v[0, off_slice] = a1
        it_v[1, off_slice] = a2
        return carry

    lax.fori_loop(0, _GRP, grp, 0)

    pltpu.sync_copy(pt_v, probs_t_hbm.at[:, pl.ds(base, _TPW)])
    pltpu.sync_copy(it_v, idx_t_hbm.at[:, pl.ds(base, _TPW)])


@functools.cache
def _make_routing():
    return pl.kernel(
        _routing_body,
        mesh=plsc.VectorSubcoreMesh(core_axis_name="c", subcore_axis_name="s"),
        out_type=[
            jax.ShapeDtypeStruct((_NE, _N_TOK), jnp.float32),
            jax.ShapeDtypeStruct((2, _N_TOK), jnp.int32),
        ],
        scratch_types=[
            pltpu.VMEM((_NE, _TPW), jnp.float32),
            pltpu.VMEM((_NE, _TPW), jnp.float32),
            pltpu.VMEM((2, _TPW), jnp.int32),
        ],
        compiler_params=pltpu.CompilerParams(needs_layout_passes=False),
    )


def kernel(x, gate_w, gate_b, noise_w, noise_b):
    logits_t = _compute_logits_t(x, gate_w, gate_b)
    probs_t, idx_t = _make_routing()(logits_t)
    return probs_t.T, idx_t.T


# speedup vs baseline: 1.0553x; 1.0553x over previous
import functools

import jax
import jax.numpy as jnp
from jax import lax
from jax.experimental import pallas as pl
from jax.experimental.pallas import tpu as pltpu
from jax.experimental.pallas import tpu_sc as plsc

_N_TOK = 16384
_D = 2048
_NE = 16
_TOK_BLK = 1024

_NW = 32
_TPW = _N_TOK // _NW
_GRP = _TPW // 16


def _logits_body(x_ref, w_ref, b_ref, o_ref):
    o_ref[...] = lax.dot_general(
        w_ref[...], x_ref[...], (((1,), (1,)), ((), ())),
        preferred_element_type=jnp.float32,
    ) + jnp.transpose(b_ref[...])


def _compute_logits_t(x, gate_w, gate_b):
    nb = _N_TOK // _TOK_BLK
    return pl.pallas_call(
        _logits_body,
        grid=(nb,),
        in_specs=[
            pl.BlockSpec((_TOK_BLK, _D), lambda i: (i, 0)),
            pl.BlockSpec((_NE, _D), lambda i: (0, 0)),
            pl.BlockSpec((1, _NE), lambda i: (0, 0)),
        ],
        out_specs=pl.BlockSpec((_NE, _TOK_BLK), lambda i: (0, i)),
        out_shape=jax.ShapeDtypeStruct((_NE, _N_TOK), jnp.float32),
    )(x, gate_w, gate_b.reshape(1, _NE))


def _routing_body(lt_hbm, probs_t_hbm, idx_t_hbm, lt_v, pt_v, it_v):
    c = lax.axis_index("c")
    s = lax.axis_index("s")
    wid = s * 2 + c
    base = wid * _TPW
    pltpu.sync_copy(lt_hbm.at[:, pl.ds(base, _TPW)], lt_v)

    neg_inf = jnp.full((16,), -jnp.inf, jnp.float32)

    def grp(g, carry):
        off = g * 16
        rows = [lt_v[e, pl.ds(off, 16)] for e in range(_NE)]
        m1 = rows[0]
        a1 = jnp.zeros((16,), jnp.int32)
        for e in range(1, _NE):
            upd = rows[e] > m1
            m1 = jnp.where(upd, rows[e], m1)
            a1 = jnp.where(upd, e, a1)
        m2 = neg_inf
        a2 = jnp.zeros((16,), jnp.int32)
        for e in range(_NE):
            v = jnp.where(a1 == e, neg_inf, rows[e])
            upd = v > m2
            m2 = jnp.where(upd, v, m2)
            a2 = jnp.where(upd, e, a2)
        t = jnp.exp(m2 - m1)
        denom = 1.0 + t
        p1 = 1.0 / denom
        p2 = t / denom
        off_slice = pl.ds(off, 16)
        zeros_f = jnp.zeros((16,), jnp.float32)
        for e in range(_NE):
            pt_v[e, off_slice] = (jnp.where(a1 == e, p1, zeros_f)
                                  + jnp.where(a2 == e, p2, zeros_f))
        it_v[0, off_slice] = a1
        it_v[1, off_slice] = a2
        return carry

    lax.fori_loop(0, _GRP, grp, 0)

    pltpu.sync_copy(pt_v, probs_t_hbm.at[:, pl.ds(base, _TPW)])
    pltpu.sync_copy(it_v, idx_t_hbm.at[:, pl.ds(base, _TPW)])


@functools.cache
def _make_routing():
    return pl.kernel(
        _routing_body,
        mesh=plsc.VectorSubcoreMesh(core_axis_name="c", subcore_axis_name="s"),
        out_type=[
            jax.ShapeDtypeStruct((_NE, _N_TOK), jnp.float32),
            jax.ShapeDtypeStruct((2, _N_TOK), jnp.int32),
        ],
        scratch_types=[
            pltpu.VMEM((_NE, _TPW), jnp.float32),
            pltpu.VMEM((_NE, _TPW), jnp.float32),
            pltpu.VMEM((2, _TPW), jnp.int32),
        ],
        compiler_params=pltpu.CompilerParams(needs_layout_passes=False),
    )


def kernel(x, gate_w, gate_b, noise_w, noise_b):
    logits_t = _compute_logits_t(x, gate_w, gate_b)
    probs_t, idx_t = _make_routing()(logits_t)
    return probs_t.T, idx_t.T


# speedup vs baseline: 1.0727x; 1.0165x over previous
import functools

import jax
import jax.numpy as jnp
from jax import lax
from jax.experimental import pallas as pl
from jax.experimental.pallas import tpu as pltpu
from jax.experimental.pallas import tpu_sc as plsc

_N_TOK = 16384
_D = 2048
_NE = 16
_TOK_BLK = 1024

_NW = 32
_TPW = _N_TOK // _NW
_GRP = _TPW // 16


def _logits_body(x_ref, w_ref, b_ref, o_ref):
    o_ref[...] = lax.dot_general(
        w_ref[...], x_ref[...], (((1,), (1,)), ((), ())),
        preferred_element_type=jnp.float32,
    ) + jnp.transpose(b_ref[...])


def _compute_logits_t(x, gate_w, gate_b):
    nb = _N_TOK // _TOK_BLK
    return pl.pallas_call(
        _logits_body,
        grid=(nb,),
        in_specs=[
            pl.BlockSpec((_TOK_BLK, _D), lambda i: (i, 0)),
            pl.BlockSpec((_NE, _D), lambda i: (0, 0)),
            pl.BlockSpec((1, _NE), lambda i: (0, 0)),
        ],
        out_specs=pl.BlockSpec((_NE, _TOK_BLK), lambda i: (0, i)),
        out_shape=jax.ShapeDtypeStruct((_NE, _N_TOK), jnp.float32),
    )(x, gate_w, gate_b.reshape(1, _NE))


def _routing_body(lt_hbm, probs_t_hbm, idx_t_hbm, lt_v, pt_v, it_v):
    c = lax.axis_index("c")
    s = lax.axis_index("s")
    wid = s * 2 + c
    base = wid * _TPW
    pltpu.sync_copy(lt_hbm.at[:, pl.ds(base, _TPW)], lt_v)

    neg_inf = jnp.full((16,), -jnp.inf, jnp.float32)
    zeros_f = jnp.zeros((16,), jnp.float32)
    lanes = lax.iota(jnp.int32, 16)

    def zero(i, carry):
        for e in range(_NE):
            pt_v[e, pl.ds(i * 16, 16)] = zeros_f
        return carry

    lax.fori_loop(0, _GRP, zero, 0)

    def grp(g, carry):
        off = g * 16
        rows = [lt_v[e, pl.ds(off, 16)] for e in range(_NE)]
        m1 = rows[0]
        a1 = jnp.zeros((16,), jnp.int32)
        for e in range(1, _NE):
            upd = rows[e] > m1
            m1 = jnp.where(upd, rows[e], m1)
            a1 = jnp.where(upd, e, a1)
        m2 = neg_inf
        a2 = jnp.zeros((16,), jnp.int32)
        for e in range(_NE):
            v = jnp.where(a1 == e, neg_inf, rows[e])
            upd = v > m2
            m2 = jnp.where(upd, v, m2)
            a2 = jnp.where(upd, e, a2)
        t = jnp.exp(m2 - m1)
        denom = 1.0 + t
        p1 = 1.0 / denom
        p2 = t / denom
        off_slice = pl.ds(off, 16)
        toks = off + lanes
        plsc.store_scatter(pt_v, [a1, toks], p1)
        plsc.store_scatter(pt_v, [a2, toks], p2)
        it_v[0, off_slice] = a1
        it_v[1, off_slice] = a2
        return carry

    lax.fori_loop(0, _GRP, grp, 0)

    pltpu.sync_copy(pt_v, probs_t_hbm.at[:, pl.ds(base, _TPW)])
    pltpu.sync_copy(it_v, idx_t_hbm.at[:, pl.ds(base, _TPW)])


@functools.cache
def _make_routing():
    return pl.kernel(
        _routing_body,
        mesh=plsc.VectorSubcoreMesh(core_axis_name="c", subcore_axis_name="s"),
        out_type=[
            jax.ShapeDtypeStruct((_NE, _N_TOK), jnp.float32),
            jax.ShapeDtypeStruct((2, _N_TOK), jnp.int32),
        ],
        scratch_types=[
            pltpu.VMEM((_NE, _TPW), jnp.float32),
            pltpu.VMEM((_NE, _TPW), jnp.float32),
            pltpu.VMEM((2, _TPW), jnp.int32),
        ],
        compiler_params=pltpu.CompilerParams(needs_layout_passes=False),
    )


def kernel(x, gate_w, gate_b, noise_w, noise_b):
    logits_t = _compute_logits_t(x, gate_w, gate_b)
    probs_t, idx_t = _make_routing()(logits_t)
    return probs_t.T, idx_t.T
